# baseline (speedup 1.0000x reference)
import functools

import jax
import jax.numpy as jnp
from jax import lax
from jax.experimental import pallas as pl
from jax.experimental.pallas import tpu as pltpu
from jax.experimental.pallas import tpu_sc as plsc

B = 16384
F = 32
N = 1000000
NC = 2
NS = 16
L = 16
NW = NC * NS
BPW = B // NW
HALF = BPW // 2
CH = 128
RPL = 4
LINES = N // RPL

CNJ = 2
CID = 128 * CNJ
CW = CID * F
NFULL = N // CID
KPW = NFULL // NW
NTAIL = N - NFULL * CID


def _relayout_body(ut_hbm, it_hbm, utail_hbm, itail_hbm, u1_hbm, i1_hbm,
                   tb0_v, tb1_v, tb2_v, tb3_v,
                   ob0_v, ob1_v, ob2_v, ob3_v, isems, osems):
  tb = (tb0_v, tb1_v, tb2_v, tb3_v)
  ob = (ob0_v, ob1_v, ob2_v, ob3_v)
  wid = lax.axis_index("s") * NC + lax.axis_index("c")
  iota_f = lax.iota(jnp.int32, L) * F

  def c0_of(k):
    return (k * NW + wid) * CID

  def issue_in(k, b):
    c0 = c0_of(k)
    for t, tbl in enumerate((ut_hbm, it_hbm)):
      for g in range(4):
        pltpu.async_copy(tbl.at[g, :, pl.ds(c0, CID)],
                         tb[2 * b + t].at[pl.ds(8 * g, 8), :], isems.at[b])

  def wait_in(b):
    for t in range(2):
      for g in range(4):
        pltpu.make_async_copy(ut_hbm.at[0, :, pl.ds(0, CID)],
                              tb[2 * b + t].at[pl.ds(8 * 0, 8), :],
                              isems.at[b]).wait()

  def compute(b):
    def grp(q, _):
      c0 = q * L
      for t in range(2):
        base = iota_f + c0 * F
        for f in range(F):
          vals = tb[2 * b + t][f, pl.ds(c0, L)]
          plsc.store_scatter(ob[2 * b + t], [base + f], vals)
      return 0
    lax.fori_loop(0, CID // L, grp, 0)

  def issue_out(k, b):
    w0 = c0_of(k) * F
    pltpu.async_copy(ob[2 * b + 0], u1_hbm.at[pl.ds(w0, CW)], osems.at[b])
    pltpu.async_copy(ob[2 * b + 1], i1_hbm.at[pl.ds(w0, CW)], osems.at[b])

  def wait_out(b):
    for t in range(2):
      pltpu.make_async_copy(ob[2 * b + t], u1_hbm.at[pl.ds(0, CW)],
                            osems.at[b]).wait()

  issue_in(0, 0)
  issue_in(1, 1)

  def outer(m, _):
    for b in range(2):
      k = 2 * m + b
      wait_in(b)

      @pl.when(m > 0)
      def _():
        wait_out(b)

      compute(b)
      issue_out(k, b)

      @pl.when(k + 2 < KPW)
      def _():
        issue_in(k + 2, b)
    return 0

  lax.fori_loop(0, KPW // 2, outer, 0)
  wait_out(0)
  wait_out(1)

  @pl.when(wid < NFULL - KPW * NW)
  def _():
    k_extra = KPW * NW + wid
    c0 = k_extra * CID
    for t, tbl in enumerate((ut_hbm, it_hbm)):
      for g in range(4):
        pltpu.async_copy(tbl.at[g, :, pl.ds(c0, CID)],
                         tb[t].at[pl.ds(8 * g, 8), :], isems.at[0])
    for t in range(2):
      for g in range(4):
        pltpu.make_async_copy(ut_hbm.at[0, :, pl.ds(0, CID)],
                              tb[t].at[pl.ds(8 * 0, 8), :],
                              isems.at[0]).wait()

    def grp(q, _):
      c0q = q * L
      for t in range(2):
        base = iota_f + c0q * F
        for f in range(F):
          vals = tb[t][f, pl.ds(c0q, L)]
          plsc.store_scatter(ob[t], [base + f], vals)
      return 0
    lax.fori_loop(0, CID // L, grp, 0)
    w0 = c0 * F
    pltpu.async_copy(ob[0], u1_hbm.at[pl.ds(w0, CW)], osems.at[0])
    pltpu.async_copy(ob[1], i1_hbm.at[pl.ds(w0, CW)], osems.at[0])
    for t in range(2):
      pltpu.make_async_copy(ob[t], u1_hbm.at[pl.ds(0, CW)],
                            osems.at[0]).wait()

  @pl.when(wid == 2)
  def _():
    w0 = NFULL * CID * F
    pltpu.sync_copy(utail_hbm, ob[0].at[pl.ds(0, NTAIL * F)])
    pltpu.sync_copy(itail_hbm, ob[1].at[pl.ds(0, NTAIL * F)])
    pltpu.sync_copy(ob[0].at[pl.ds(0, NTAIL * F)],
                    u1_hbm.at[pl.ds(w0, NTAIL * F)])
    pltpu.sync_copy(ob[1].at[pl.ds(0, NTAIL * F)],
                    i1_hbm.at[pl.ds(w0, NTAIL * F)])


_relayout = functools.partial(
    pl.kernel,
    out_type=(jax.ShapeDtypeStruct((N * F,), jnp.float32),
              jax.ShapeDtypeStruct((N * F,), jnp.float32)),
    mesh=plsc.VectorSubcoreMesh(core_axis_name="c", subcore_axis_name="s"),
    scratch_types=[
        pltpu.VMEM((F, CID), jnp.float32),
        pltpu.VMEM((F, CID), jnp.float32),
        pltpu.VMEM((F, CID), jnp.float32),
        pltpu.VMEM((F, CID), jnp.float32),
        pltpu.VMEM((CW,), jnp.float32),
        pltpu.VMEM((CW,), jnp.float32),
        pltpu.VMEM((CW,), jnp.float32),
        pltpu.VMEM((CW,), jnp.float32),
        pltpu.SemaphoreType.DMA((2,)),
        pltpu.SemaphoreType.DMA((2,)),
    ],
    compiler_params=pltpu.CompilerParams(
        needs_layout_passes=False, use_tc_tiling_on_sc=True),
)(_relayout_body)


def _mf_dot_body(uid_hbm, iid_hbm, ut_hbm, it_hbm, out_hbm,
                 uidx_v, iidx_v, ulidx_v, ilidx_v,
                 urows_v, irows_v, out_v, sem):
  wid = lax.axis_index("s") * NC + lax.axis_index("c")
  base = wid * BPW

  pltpu.sync_copy(uid_hbm.at[pl.ds(base, BPW)], uidx_v)
  pltpu.sync_copy(iid_hbm.at[pl.ds(base, BPW)], iidx_v)

  def mkline(g, _):
    sl = pl.ds(g * L, L)
    ulidx_v[sl] = lax.shift_right_logical(uidx_v[sl], 2)
    ilidx_v[sl] = lax.shift_right_logical(iidx_v[sl], 2)
    return 0

  lax.fori_loop(0, BPW // L, mkline, 0)

  iota = lax.iota(jnp.int32, L)

  for h in range(BPW // HALF):
    copies = []
    for j in range(HALF // CH):
      isl = pl.ds(h * HALF + j * CH, CH)
      dsl = pl.ds(j * CH, CH)
      copies.append(pltpu.async_copy(
          ut_hbm.at[ulidx_v.at[isl]], urows_v.at[dsl], sem))
      copies.append(pltpu.async_copy(
          it_hbm.at[ilidx_v.at[isl]], irows_v.at[dsl], sem))
    for c in copies:
      c.wait()

    def body(g, _):
      sl = pl.ds(h * HALF + g * L, L)
      ucol = (uidx_v[sl] & 3) * F
      icol = (iidx_v[sl] & 3) * F
      row = g * L + iota
      acc = jnp.zeros((L,), jnp.float32)
      for f in range(F):
        u = plsc.load_gather(urows_v, [row, ucol + f])
        v = plsc.load_gather(irows_v, [row, icol + f])
        acc = acc + u * v
      out_v[sl] = acc
      return 0

    lax.fori_loop(0, HALF // L, body, 0)

  pltpu.sync_copy(out_v, out_hbm.at[pl.ds(base, BPW)])


_mf_dot = functools.partial(
    pl.kernel,
    out_type=jax.ShapeDtypeStruct((B,), jnp.float32),
    mesh=plsc.VectorSubcoreMesh(core_axis_name="c", subcore_axis_name="s"),
    scratch_types=[
        pltpu.VMEM((BPW,), jnp.int32),
        pltpu.VMEM((BPW,), jnp.int32),
        pltpu.VMEM((BPW,), jnp.int32),
        pltpu.VMEM((BPW,), jnp.int32),
        pltpu.VMEM((HALF, RPL * F), jnp.float32),
        pltpu.VMEM((HALF, RPL * F), jnp.float32),
        pltpu.VMEM((BPW,), jnp.float32),
        pltpu.SemaphoreType.DMA,
    ],
    compiler_params=pltpu.CompilerParams(
        needs_layout_passes=False, use_tc_tiling_on_sc=False),
)(_mf_dot_body)


@jax.jit
def kernel(user_ids, item_ids, user_table, item_table):
  u1, i1 = _relayout(user_table.T.reshape(4, 8, N),
                     item_table.T.reshape(4, 8, N),
                     user_table[NFULL * CID:].reshape(NTAIL * F),
                     item_table[NFULL * CID:].reshape(NTAIL * F))
  return _mf_dot(user_ids.astype(jnp.int32), item_ids.astype(jnp.int32),
                 u1.reshape(LINES, RPL * F), i1.reshape(LINES, RPL * F))
